# baseline (speedup 1.0000x reference)
import functools

import jax
import jax.numpy as jnp
from jax import lax
from jax.experimental import pallas as pl
from jax.experimental.pallas import tpu as pltpu
from jax.experimental.pallas import tpu_sc as plsc

N = 10000
E = 320000
DIN = 128
DHID = 128
DOUT = 64
G = 128

NP = 10240
NC = 2
NS = 16
NW = NC * NS
CH = 128
NCHUNK = 80
EP = NW * NCHUNK * CH
NCH2 = EP // (NS * CH)
RPT = NP // NS
DDEG = 16


def _get_mesh():
    return plsc.VectorSubcoreMesh(core_axis_name="c", subcore_axis_name="s",
                                  num_cores=NC, num_subcores=NS)


def _make_edge_scatter_cs(D):
    @functools.partial(
        pl.kernel,
        out_type=jax.ShapeDtypeStruct((NC * NP, D), jnp.float32),
        mesh=_get_mesh(),
        compiler_params=pltpu.CompilerParams(use_tc_tiling_on_sc=False),
        scratch_types=[
            pltpu.VMEM((NCH2, CH), jnp.int32),
            pltpu.VMEM((NCH2, CH), jnp.int32),
            pltpu.VMEM((1, CH), jnp.int32),
            pltpu.VMEM((CH, D), jnp.float32),
            pltpu.VMEM((CH, D), jnp.float32),
            pltpu.VMEM_SHARED((NP, D), jnp.float32),
            pltpu.SemaphoreType.DMA,
            pltpu.SemaphoreType.DMA,
            pltpu.SemaphoreType.DMA,
            pltpu.SemaphoreType.DMA,
        ],
    )
    def k(src_hbm, dst_hbm, y_hbm, out_hbm, src_v, dst_v, garb_v, rows0,
          rows1, acc, sg0, sg1, ss0, ss1):
        c = lax.axis_index("c")
        s = lax.axis_index("s")
        pltpu.sync_copy(src_hbm.at[c, s], src_v)
        pltpu.sync_copy(dst_hbm.at[s], dst_v)

        zv = jnp.zeros((16,), jnp.float32)
        gv = jnp.full((16,), N, jnp.int32)
        for cc in range(CH // 16):
            garb_v[0, pl.ds(cc * 16, 16)] = gv

        def zbody(r, carry):
            for cc in range(D // 16):
                rows0[r, pl.ds(cc * 16, 16)] = zv
                rows1[r, pl.ds(cc * 16, 16)] = zv
            return carry

        lax.fori_loop(0, CH, zbody, 0)
        for p in range(RPT // CH):
            pltpu.sync_copy(rows0, acc.at[pl.ds(s * RPT + p * CH, CH)])
        plsc.subcore_barrier()

        pltpu.async_copy(rows1, acc.at[dst_v.at[0]], ss1, add=True)
        pltpu.async_copy(y_hbm.at[src_v.at[0]], rows0, sg0)

        def body(k2, carry):
            a = 2 * k2
            pltpu.make_async_copy(y_hbm.at[src_v.at[a]], rows0, sg0).wait()
            pltpu.async_copy(rows0, acc.at[dst_v.at[a]], ss0, add=True)
            pltpu.make_async_copy(rows1, acc.at[dst_v.at[0]], ss1).wait()
            pltpu.async_copy(y_hbm.at[src_v.at[a + 1]], rows1, sg1)
            pltpu.make_async_copy(y_hbm.at[src_v.at[0]], rows1, sg1).wait()
            pltpu.async_copy(rows1, acc.at[dst_v.at[a + 1]], ss1, add=True)
            pltpu.make_async_copy(rows0, acc.at[dst_v.at[0]], ss0).wait()
            nxt = lax.rem(a + 2, NCH2)
            pltpu.async_copy(y_hbm.at[src_v.at[nxt]], rows0, sg0)
            return carry

        lax.fori_loop(0, NCH2 // 2, body, 0)
        pltpu.make_async_copy(y_hbm.at[src_v.at[0]], rows0, sg0).wait()
        pltpu.make_async_copy(rows1, acc.at[dst_v.at[0]], ss1).wait()
        plsc.subcore_barrier()
        pltpu.sync_copy(acc.at[pl.ds(s * RPT, RPT)],
                        out_hbm.at[pl.ds(c * NP + s * RPT, RPT)])

    return k


def _make_deg_kernel():
    @functools.partial(
        pl.kernel,
        out_type=jax.ShapeDtypeStruct((NC, NP, DDEG), jnp.float32),
        mesh=_get_mesh(),
        compiler_params=pltpu.CompilerParams(use_tc_tiling_on_sc=False),
        scratch_types=[
            pltpu.VMEM((NCHUNK, CH), jnp.int32),
            pltpu.VMEM((CH, DDEG), jnp.float32),
            pltpu.VMEM((CH, DDEG), jnp.float32),
            pltpu.VMEM_SHARED((NP, DDEG), jnp.float32),
        ],
    )
    def k(dst_hbm, ones_hbm, zero_hbm, out_hbm, dst_v, ones_v, zero_v, acc):
        c = lax.axis_index("c")
        s = lax.axis_index("s")
        wid = c * NS + s
        pltpu.sync_copy(dst_hbm.at[wid], dst_v)
        pltpu.sync_copy(ones_hbm, ones_v)
        pltpu.sync_copy(zero_hbm, zero_v)
        for p in range(RPT // CH):
            pltpu.sync_copy(zero_v, acc.at[pl.ds(s * RPT + p * CH, CH)])
        plsc.subcore_barrier()

        def body(j, carry):
            pltpu.sync_copy(ones_v, acc.at[dst_v.at[j]], add=True)
            return carry

        lax.fori_loop(0, NCHUNK, body, 0)
        plsc.subcore_barrier()
        pltpu.sync_copy(acc.at[pl.ds(s * RPT, RPT)],
                        out_hbm.at[c, pl.ds(s * RPT, RPT)])

    return k


_BR = 512
_NB = NP // _BR


def _dinv_block(d0_ref, d1_ref):
    deg = d0_ref[:, 0:1] + d1_ref[:, 0:1] + 1.0
    return lax.rsqrt(deg)


def _tc_y1(x, W1s, d0, d1):
    H = DHID // 2

    def body(x_ref, w_ref, d0_ref, d1_ref, y_ref):
        dinv = _dinv_block(d0_ref, d1_ref)
        y_ref[...] = dinv * jnp.dot(x_ref[...], w_ref[0],
                                    preferred_element_type=jnp.float32)

    return pl.pallas_call(
        body,
        grid=(_NB, 2),
        in_specs=[
            pl.BlockSpec((_BR, DIN), lambda i, j: (i, 0)),
            pl.BlockSpec((1, DIN, H), lambda i, j: (j, 0, 0)),
            pl.BlockSpec((_BR, DDEG), lambda i, j: (i, 0)),
            pl.BlockSpec((_BR, DDEG), lambda i, j: (i, 0)),
        ],
        out_specs=pl.BlockSpec((_BR, H), lambda i, j: (j * _NB + i, 0)),
        out_shape=jax.ShapeDtypeStruct((NC * NP, H), jnp.float32),
    )(x, W1s, d0, d1)


def _tc_y2(s1f, y1f, d0, d1, b1, W2s):
    H1 = DHID // 2
    H2 = DOUT // 2

    def body(s1a_ref, s1b_ref, y1a_ref, y1b_ref, d0_ref, d1_ref, b1_ref,
             w_ref, y2_ref):
        dinv = _dinv_block(d0_ref, d1_ref)
        t = jnp.concatenate(
            [s1a_ref[...] + y1a_ref[...], s1b_ref[...] + y1b_ref[...]],
            axis=1)
        h = jax.nn.relu(dinv * t + b1_ref[...])
        y2_ref[...] = dinv * jnp.dot(h, w_ref[0],
                                     preferred_element_type=jnp.float32)

    return pl.pallas_call(
        body,
        grid=(_NB, 2),
        in_specs=[
            pl.BlockSpec((_BR, H1), lambda i, j: (i, 0)),
            pl.BlockSpec((_BR, H1), lambda i, j: (_NB + i, 0)),
            pl.BlockSpec((_BR, H1), lambda i, j: (i, 0)),
            pl.BlockSpec((_BR, H1), lambda i, j: (_NB + i, 0)),
            pl.BlockSpec((_BR, DDEG), lambda i, j: (i, 0)),
            pl.BlockSpec((_BR, DDEG), lambda i, j: (i, 0)),
            pl.BlockSpec((1, DHID), lambda i, j: (0, 0)),
            pl.BlockSpec((1, DHID, H2), lambda i, j: (j, 0, 0)),
        ],
        out_specs=pl.BlockSpec((_BR, H2), lambda i, j: (j * _NB + i, 0)),
        out_shape=jax.ShapeDtypeStruct((NC * NP, H2), jnp.float32),
    )(s1f, s1f, y1f, y1f, d0, d1, b1, W2s)


def _tc_final(s2f, y2f, d0, d1, b2, batch2d, Wlp, bl2d):
    H2 = DOUT // 2

    def body(s2a_ref, s2b_ref, y2a_ref, y2b_ref, d0_ref, d1_ref, b2_ref,
             bt_ref, wl_ref, bl_ref, out_ref, acc_ref):
        i = pl.program_id(0)

        @pl.when(i == 0)
        def _():
            acc_ref[...] = jnp.zeros_like(acc_ref)

        dinv = _dinv_block(d0_ref, d1_ref)
        t = jnp.concatenate(
            [s2a_ref[...] + y2a_ref[...], s2b_ref[...] + y2b_ref[...]],
            axis=1)
        h2 = dinv * t + b2_ref[...]
        iota = lax.broadcasted_iota(jnp.int32, (G, _BR), 0)
        oh = (bt_ref[...] == iota).astype(jnp.float32)
        acc_ref[:, 0:DOUT] = acc_ref[:, 0:DOUT] + jnp.dot(
            oh, h2, preferred_element_type=jnp.float32)
        acc_ref[:, DOUT:DOUT + 1] = (acc_ref[:, DOUT:DOUT + 1]
                                     + jnp.sum(oh, axis=1, keepdims=True))

        @pl.when(i == _NB - 1)
        def _():
            cnt = jnp.maximum(acc_ref[:, DOUT:DOUT + 1], 1.0)
            g = acc_ref[:, 0:DOUT] / cnt
            out_ref[...] = jnp.dot(g, wl_ref[...],
                                   preferred_element_type=jnp.float32) \
                + bl_ref[0, 0]

    return pl.pallas_call(
        body,
        grid=(_NB,),
        in_specs=[
            pl.BlockSpec((_BR, H2), lambda i: (i, 0)),
            pl.BlockSpec((_BR, H2), lambda i: (_NB + i, 0)),
            pl.BlockSpec((_BR, H2), lambda i: (i, 0)),
            pl.BlockSpec((_BR, H2), lambda i: (_NB + i, 0)),
            pl.BlockSpec((_BR, DDEG), lambda i: (i, 0)),
            pl.BlockSpec((_BR, DDEG), lambda i: (i, 0)),
            pl.BlockSpec((1, DOUT), lambda i: (0, 0)),
            pl.BlockSpec((1, _BR), lambda i: (0, i)),
            pl.BlockSpec((DOUT, 128), lambda i: (0, 0)),
            pl.BlockSpec((1, 1), lambda i: (0, 0)),
        ],
        out_specs=pl.BlockSpec((G, 128), lambda i: (0, 0)),
        out_shape=jax.ShapeDtypeStruct((G, 128), jnp.float32),
        scratch_shapes=[pltpu.VMEM((G, 128), jnp.float32)],
    )(s2f, s2f, y2f, y2f, d0, d1, b2, batch2d, Wlp, bl2d)


_sc_cache = {}


def _deg_kernel(dst_slab):
    if "deg" not in _sc_cache:
        _sc_cache["deg"] = _make_deg_kernel()
    ones = jnp.ones((CH, DDEG), jnp.float32)
    zero = jnp.zeros((CH, DDEG), jnp.float32)
    return _sc_cache["deg"](dst_slab, ones, zero)


def _scatter_l1(src_slab, dst_slab, y):
    if 1 not in _sc_cache:
        _sc_cache[1] = _make_edge_scatter_cs(DHID // 2)
    return _sc_cache[1](src_slab, dst_slab, y)


def _scatter_l2(src_slab, dst_slab, y):
    if 2 not in _sc_cache:
        _sc_cache[2] = _make_edge_scatter_cs(DOUT // 2)
    return _sc_cache[2](src_slab, dst_slab, y)


def kernel(x, edge_index, batch, W1, b1, W2, b2, Wl, bl):
    src = edge_index[0]
    dst = edge_index[1]
    pad = jnp.full((EP - E,), N, dtype=jnp.int32)
    srcp = jnp.concatenate([src, pad])
    dstp = jnp.concatenate([dst, pad])
    dst_deg_slab = dstp.reshape(NW, NCHUNK, CH)
    src_base = srcp.reshape(NS, NCH2, CH)
    src_slab = jnp.stack([src_base, src_base + NP])
    dst_slab = dstp.reshape(NS, NCH2, CH)

    xp = jnp.zeros((NP, DIN), jnp.float32).at[:N].set(x)
    batchp = jnp.full((NP,), G, jnp.int32).at[:N].set(batch).reshape(1, NP)
    W1s = jnp.stack([W1[:, :DHID // 2], W1[:, DHID // 2:]])
    W2s = jnp.stack([W2[:, :DOUT // 2], W2[:, DOUT // 2:]])

    deg = _deg_kernel(dst_deg_slab)
    d0, d1 = deg[0], deg[1]

    y1f = _tc_y1(xp, W1s, d0, d1)
    s1f = _scatter_l1(src_slab, dst_slab, y1f)
    y2f = _tc_y2(s1f, y1f, d0, d1, b1.reshape(1, DHID), W2s)
    s2f = _scatter_l2(src_slab, dst_slab, y2f)

    Wlp = jnp.zeros((DOUT, 128), jnp.float32).at[:, 0].set(Wl[:, 0])
    out2 = _tc_final(s2f, y2f, d0, d1, b2.reshape(1, DOUT), batchp,
                     Wlp, bl.reshape(1, 1))
    return out2[:, 0:1]


# speedup vs baseline: 1.1211x; 1.1211x over previous
import functools

import jax
import jax.numpy as jnp
from jax import lax
from jax.experimental import pallas as pl
from jax.experimental.pallas import tpu as pltpu
from jax.experimental.pallas import tpu_sc as plsc

N = 10000
E = 320000
DIN = 128
DHID = 128
DOUT = 64
G = 128

NP = 10240
NC = 2
NS = 16
NW = NC * NS
CH = 128
NCHUNK = 80
EP = NW * NCHUNK * CH
NCH2 = EP // (NS * CH)
RPT = NP // NS
DDEG = 16


def _get_mesh():
    return plsc.VectorSubcoreMesh(core_axis_name="c", subcore_axis_name="s",
                                  num_cores=NC, num_subcores=NS)


def _make_edge_scatter_cs(D):
    @functools.partial(
        pl.kernel,
        out_type=jax.ShapeDtypeStruct((NC * NP, D), jnp.float32),
        mesh=_get_mesh(),
        compiler_params=pltpu.CompilerParams(use_tc_tiling_on_sc=False),
        scratch_types=[
            pltpu.VMEM((NCH2, CH), jnp.int32),
            pltpu.VMEM((NCH2, CH), jnp.int32),
            pltpu.VMEM((CH, D), jnp.float32),
            pltpu.VMEM((CH, D), jnp.float32),
            pltpu.VMEM_SHARED((NP, D), jnp.float32),
            pltpu.SemaphoreType.DMA,
            pltpu.SemaphoreType.DMA,
            pltpu.SemaphoreType.DMA,
            pltpu.SemaphoreType.DMA,
        ],
    )
    def k(src_hbm, dst_hbm, y_hbm, out_hbm, src_v, dst_v, rows0,
          rows1, acc, sg0, sg1, ss0, ss1):
        c = lax.axis_index("c")
        s = lax.axis_index("s")
        pltpu.sync_copy(src_hbm.at[c, s], src_v)
        pltpu.sync_copy(dst_hbm.at[s], dst_v)

        zv = jnp.zeros((16,), jnp.float32)

        def zbody(r, carry):
            for cc in range(D // 16):
                rows0[r, pl.ds(cc * 16, 16)] = zv
                rows1[r, pl.ds(cc * 16, 16)] = zv
            return carry

        lax.fori_loop(0, CH, zbody, 0)
        for p in range(RPT // CH):
            pltpu.sync_copy(rows0, acc.at[pl.ds(s * RPT + p * CH, CH)])
        plsc.subcore_barrier()

        pltpu.async_copy(rows1, acc.at[dst_v.at[0]], ss1, add=True)
        pltpu.async_copy(y_hbm.at[src_v.at[0]], rows0, sg0)

        def body(k2, carry):
            a = 2 * k2
            pltpu.make_async_copy(y_hbm.at[src_v.at[a]], rows0, sg0).wait()
            pltpu.async_copy(rows0, acc.at[dst_v.at[a]], ss0, add=True)
            pltpu.make_async_copy(rows1, acc.at[dst_v.at[0]], ss1).wait()
            pltpu.async_copy(y_hbm.at[src_v.at[a + 1]], rows1, sg1)
            pltpu.make_async_copy(y_hbm.at[src_v.at[0]], rows1, sg1).wait()
            pltpu.async_copy(rows1, acc.at[dst_v.at[a + 1]], ss1, add=True)
            pltpu.make_async_copy(rows0, acc.at[dst_v.at[0]], ss0).wait()
            nxt = lax.rem(a + 2, NCH2)
            pltpu.async_copy(y_hbm.at[src_v.at[nxt]], rows0, sg0)
            return carry

        lax.fori_loop(0, NCH2 // 2, body, 0)
        pltpu.make_async_copy(y_hbm.at[src_v.at[0]], rows0, sg0).wait()
        pltpu.make_async_copy(rows1, acc.at[dst_v.at[0]], ss1).wait()
        plsc.subcore_barrier()
        pltpu.sync_copy(acc.at[pl.ds(s * RPT, RPT)],
                        out_hbm.at[pl.ds(c * NP + s * RPT, RPT)])

    return k


def _make_deg_kernel():
    @functools.partial(
        pl.kernel,
        out_type=jax.ShapeDtypeStruct((NC, NP, DDEG), jnp.float32),
        mesh=_get_mesh(),
        scratch_types=[
            pltpu.VMEM((NCHUNK, CH), jnp.int32),
            pltpu.VMEM((CH, DDEG), jnp.float32),
            pltpu.VMEM((CH, DDEG), jnp.float32),
            pltpu.VMEM_SHARED((NP, DDEG), jnp.float32),
        ],
    )
    def k(dst_hbm, out_hbm, dst_v, ones_v, zero_v, acc):
        c = lax.axis_index("c")
        s = lax.axis_index("s")
        wid = c * NS + s
        pltpu.sync_copy(dst_hbm.at[wid], dst_v)

        ov = jnp.full((16,), 1.0, jnp.float32)
        zv = jnp.zeros((16,), jnp.float32)

        def fbody(r, carry):
            ones_v[r, pl.ds(0, DDEG)] = ov
            zero_v[r, pl.ds(0, DDEG)] = zv
            return carry

        lax.fori_loop(0, CH, fbody, 0)
        for p in range(RPT // CH):
            pltpu.sync_copy(zero_v, acc.at[pl.ds(s * RPT + p * CH, CH)])
        plsc.subcore_barrier()

        def body(j, carry):
            pltpu.sync_copy(ones_v, acc.at[dst_v.at[j]], add=True)
            return carry

        lax.fori_loop(0, NCHUNK, body, 0)
        plsc.subcore_barrier()
        pltpu.sync_copy(acc.at[pl.ds(s * RPT, RPT)],
                        out_hbm.at[c, pl.ds(s * RPT, RPT)])

    return k


_BR = 512
_NB = NP // _BR


def _dinv_block(d0_ref, d1_ref):
    deg = d0_ref[:, 0:1] + d1_ref[:, 0:1] + 1.0
    return lax.rsqrt(deg)


def _tc_y1(x, W1s, d0, d1):
    H = DHID // 2

    def body(x_ref, w_ref, d0_ref, d1_ref, y_ref):
        dinv = _dinv_block(d0_ref, d1_ref)
        y_ref[...] = dinv * jnp.dot(x_ref[...], w_ref[0],
                                    preferred_element_type=jnp.float32)

    return pl.pallas_call(
        body,
        grid=(_NB, 2),
        in_specs=[
            pl.BlockSpec((_BR, DIN), lambda i, j: (i, 0)),
            pl.BlockSpec((1, DIN, H), lambda i, j: (j, 0, 0)),
            pl.BlockSpec((_BR, DDEG), lambda i, j: (i, 0)),
            pl.BlockSpec((_BR, DDEG), lambda i, j: (i, 0)),
        ],
        out_specs=pl.BlockSpec((_BR, H), lambda i, j: (j * _NB + i, 0)),
        out_shape=jax.ShapeDtypeStruct((NC * NP, H), jnp.float32),
    )(x, W1s, d0, d1)


def _tc_y2(s1f, y1f, d0, d1, b1, W2s):
    H1 = DHID // 2
    H2 = DOUT // 2

    def body(s1a_ref, s1b_ref, y1a_ref, y1b_ref, d0_ref, d1_ref, b1_ref,
             w_ref, y2_ref):
        dinv = _dinv_block(d0_ref, d1_ref)
        t = jnp.concatenate(
            [s1a_ref[...] + y1a_ref[...], s1b_ref[...] + y1b_ref[...]],
            axis=1)
        h = jax.nn.relu(dinv * t + b1_ref[...])
        y2_ref[...] = dinv * jnp.dot(h, w_ref[0],
                                     preferred_element_type=jnp.float32)

    return pl.pallas_call(
        body,
        grid=(_NB, 2),
        in_specs=[
            pl.BlockSpec((_BR, H1), lambda i, j: (i, 0)),
            pl.BlockSpec((_BR, H1), lambda i, j: (_NB + i, 0)),
            pl.BlockSpec((_BR, H1), lambda i, j: (i, 0)),
            pl.BlockSpec((_BR, H1), lambda i, j: (_NB + i, 0)),
            pl.BlockSpec((_BR, DDEG), lambda i, j: (i, 0)),
            pl.BlockSpec((_BR, DDEG), lambda i, j: (i, 0)),
            pl.BlockSpec((1, DHID), lambda i, j: (0, 0)),
            pl.BlockSpec((1, DHID, H2), lambda i, j: (j, 0, 0)),
        ],
        out_specs=pl.BlockSpec((_BR, H2), lambda i, j: (j * _NB + i, 0)),
        out_shape=jax.ShapeDtypeStruct((NC * NP, H2), jnp.float32),
    )(s1f, s1f, y1f, y1f, d0, d1, b1, W2s)


def _tc_final(s2f, y2f, d0, d1, b2, batch2d, Wlp, bl2d):
    H2 = DOUT // 2

    def body(s2a_ref, s2b_ref, y2a_ref, y2b_ref, d0_ref, d1_ref, b2_ref,
             bt_ref, wl_ref, bl_ref, out_ref, acc_ref):
        i = pl.program_id(0)

        @pl.when(i == 0)
        def _():
            acc_ref[...] = jnp.zeros_like(acc_ref)

        dinv = _dinv_block(d0_ref, d1_ref)
        t = jnp.concatenate(
            [s2a_ref[...] + y2a_ref[...], s2b_ref[...] + y2b_ref[...]],
            axis=1)
        h2 = dinv * t + b2_ref[...]
        iota = lax.broadcasted_iota(jnp.int32, (G, _BR), 0)
        oh = (bt_ref[...] == iota).astype(jnp.float32)
        acc_ref[:, 0:DOUT] = acc_ref[:, 0:DOUT] + jnp.dot(
            oh, h2, preferred_element_type=jnp.float32)
        acc_ref[:, DOUT:DOUT + 1] = (acc_ref[:, DOUT:DOUT + 1]
                                     + jnp.sum(oh, axis=1, keepdims=True))

        @pl.when(i == _NB - 1)
        def _():
            cnt = jnp.maximum(acc_ref[:, DOUT:DOUT + 1], 1.0)
            g = acc_ref[:, 0:DOUT] / cnt
            out_ref[...] = jnp.dot(g, wl_ref[...],
                                   preferred_element_type=jnp.float32) \
                + bl_ref[0, 0]

    return pl.pallas_call(
        body,
        grid=(_NB,),
        in_specs=[
            pl.BlockSpec((_BR, H2), lambda i: (i, 0)),
            pl.BlockSpec((_BR, H2), lambda i: (_NB + i, 0)),
            pl.BlockSpec((_BR, H2), lambda i: (i, 0)),
            pl.BlockSpec((_BR, H2), lambda i: (_NB + i, 0)),
            pl.BlockSpec((_BR, DDEG), lambda i: (i, 0)),
            pl.BlockSpec((_BR, DDEG), lambda i: (i, 0)),
            pl.BlockSpec((1, DOUT), lambda i: (0, 0)),
            pl.BlockSpec((1, _BR), lambda i: (0, i)),
            pl.BlockSpec((DOUT, 128), lambda i: (0, 0)),
            pl.BlockSpec((1, 1), lambda i: (0, 0)),
        ],
        out_specs=pl.BlockSpec((G, 128), lambda i: (0, 0)),
        out_shape=jax.ShapeDtypeStruct((G, 128), jnp.float32),
        scratch_shapes=[pltpu.VMEM((G, 128), jnp.float32)],
    )(s2f, s2f, y2f, y2f, d0, d1, b2, batch2d, Wlp, bl2d)


_sc_cache = {}


def _deg_kernel(dst_slab):
    if "deg" not in _sc_cache:
        _sc_cache["deg"] = _make_deg_kernel()
    return _sc_cache["deg"](dst_slab)


def _scatter_l1(src_slab, dst_slab, y):
    if 1 not in _sc_cache:
        _sc_cache[1] = _make_edge_scatter_cs(DHID // 2)
    return _sc_cache[1](src_slab, dst_slab, y)


def _scatter_l2(src_slab, dst_slab, y):
    if 2 not in _sc_cache:
        _sc_cache[2] = _make_edge_scatter_cs(DOUT // 2)
    return _sc_cache[2](src_slab, dst_slab, y)


def kernel(x, edge_index, batch, W1, b1, W2, b2, Wl, bl):
    src = edge_index[0]
    dst = edge_index[1]
    pad = jnp.full((EP - E,), N, dtype=jnp.int32)
    srcp = jnp.concatenate([src, pad])
    dstp = jnp.concatenate([dst, pad])
    dst_deg_slab = dstp.reshape(NW, NCHUNK, CH)
    src_base = srcp.reshape(NS, NCH2, CH)
    src_slab = jnp.stack([src_base, src_base + NP])
    dst_slab = dstp.reshape(NS, NCH2, CH)

    xp = jnp.zeros((NP, DIN), jnp.float32).at[:N].set(x)
    batchp = jnp.full((NP,), G, jnp.int32).at[:N].set(batch).reshape(1, NP)
    W1s = jnp.stack([W1[:, :DHID // 2], W1[:, DHID // 2:]])
    W2s = jnp.stack([W2[:, :DOUT // 2], W2[:, DOUT // 2:]])

    deg = _deg_kernel(dst_deg_slab)
    d0, d1 = deg[0], deg[1]

    y1f = _tc_y1(xp, W1s, d0, d1)
    s1f = _scatter_l1(src_slab, dst_slab, y1f)
    y2f = _tc_y2(s1f, y1f, d0, d1, b1.reshape(1, DHID), W2s)
    s2f = _scatter_l2(src_slab, dst_slab, y2f)

    Wlp = jnp.zeros((DOUT, 128), jnp.float32).at[:, 0].set(Wl[:, 0])
    out2 = _tc_final(s2f, y2f, d0, d1, b2.reshape(1, DOUT), batchp,
                     Wlp, bl.reshape(1, 1))
    return out2[:, 0:1]


# speedup vs baseline: 1.2611x; 1.1249x over previous
import functools

import jax
import jax.numpy as jnp
from jax import lax
from jax.experimental import pallas as pl
from jax.experimental.pallas import tpu as pltpu
from jax.experimental.pallas import tpu_sc as plsc

N = 10000
E = 320000
DIN = 128
DHID = 128
DOUT = 64
G = 128

NP = 10240
NC = 2
NS = 16
NW = NC * NS
CH = 128
NCHUNK = 80
EP = NW * NCHUNK * CH
NCH2 = EP // (NS * CH)
RPT = NP // NS
DDEG = 16


def _get_mesh():
    return plsc.VectorSubcoreMesh(core_axis_name="c", subcore_axis_name="s",
                                  num_cores=NC, num_subcores=NS)


def _make_edge_scatter_cs(D):
    @functools.partial(
        pl.kernel,
        out_type=jax.ShapeDtypeStruct((NC * NP, D), jnp.float32),
        mesh=_get_mesh(),
        compiler_params=pltpu.CompilerParams(use_tc_tiling_on_sc=False),
        scratch_types=[
            pltpu.VMEM((NCH2, CH), jnp.int32),
            pltpu.VMEM((NCH2, CH), jnp.int32),
            pltpu.VMEM((CH, D), jnp.float32),
            pltpu.VMEM((CH, D), jnp.float32),
            pltpu.VMEM((CH, D), jnp.float32),
            pltpu.VMEM((CH, D), jnp.float32),
            pltpu.VMEM_SHARED((NP, D), jnp.float32),
            pltpu.SemaphoreType.DMA,
            pltpu.SemaphoreType.DMA,
            pltpu.SemaphoreType.DMA,
            pltpu.SemaphoreType.DMA,
            pltpu.SemaphoreType.DMA,
            pltpu.SemaphoreType.DMA,
            pltpu.SemaphoreType.DMA,
            pltpu.SemaphoreType.DMA,
        ],
    )
    def k(src_hbm, dst_hbm, y_hbm, out_hbm, src_v, dst_v, rows0,
          rows1, rows2, rows3, acc, sg0, sg1, sg2, sg3, ss0, ss1, ss2, ss3):
        c = lax.axis_index("c")
        s = lax.axis_index("s")
        pltpu.sync_copy(src_hbm.at[c, s], src_v)
        pltpu.sync_copy(dst_hbm.at[s], dst_v)

        zv = jnp.zeros((16,), jnp.float32)

        def zbody(r, carry):
            for cc in range(D // 16):
                rows0[r, pl.ds(cc * 16, 16)] = zv
                rows1[r, pl.ds(cc * 16, 16)] = zv
                rows3[r, pl.ds(cc * 16, 16)] = zv
            return carry

        lax.fori_loop(0, CH, zbody, 0)
        for p in range(RPT // CH):
            pltpu.sync_copy(rows0, acc.at[pl.ds(s * RPT + p * CH, CH)])
        plsc.subcore_barrier()

        pltpu.async_copy(rows1, acc.at[dst_v.at[0]], ss1, add=True)
        pltpu.async_copy(rows3, acc.at[dst_v.at[0]], ss3, add=True)
        pltpu.async_copy(y_hbm.at[src_v.at[0]], rows0, sg0)
        pltpu.async_copy(y_hbm.at[src_v.at[1]], rows2, sg2)

        def body(k2, carry):
            a = 4 * k2
            pltpu.make_async_copy(y_hbm.at[src_v.at[a]], rows0, sg0).wait()
            pltpu.async_copy(rows0, acc.at[dst_v.at[a]], ss0, add=True)
            pltpu.make_async_copy(y_hbm.at[src_v.at[0]], rows2, sg2).wait()
            pltpu.async_copy(rows2, acc.at[dst_v.at[a + 1]], ss2, add=True)
            pltpu.make_async_copy(rows1, acc.at[dst_v.at[0]], ss1).wait()
            pltpu.async_copy(y_hbm.at[src_v.at[a + 2]], rows1, sg1)
            pltpu.make_async_copy(rows3, acc.at[dst_v.at[0]], ss3).wait()
            pltpu.async_copy(y_hbm.at[src_v.at[a + 3]], rows3, sg3)
            pltpu.make_async_copy(y_hbm.at[src_v.at[0]], rows1, sg1).wait()
            pltpu.async_copy(rows1, acc.at[dst_v.at[a + 2]], ss1, add=True)
            pltpu.make_async_copy(y_hbm.at[src_v.at[0]], rows3, sg3).wait()
            pltpu.async_copy(rows3, acc.at[dst_v.at[a + 3]], ss3, add=True)
            pltpu.make_async_copy(rows0, acc.at[dst_v.at[0]], ss0).wait()
            na = lax.rem(a + 4, NCH2)
            pltpu.async_copy(y_hbm.at[src_v.at[na]], rows0, sg0)
            pltpu.make_async_copy(rows2, acc.at[dst_v.at[0]], ss2).wait()
            nb = lax.rem(a + 5, NCH2)
            pltpu.async_copy(y_hbm.at[src_v.at[nb]], rows2, sg2)
            return carry

        lax.fori_loop(0, NCH2 // 4, body, 0)
        pltpu.make_async_copy(y_hbm.at[src_v.at[0]], rows0, sg0).wait()
        pltpu.make_async_copy(y_hbm.at[src_v.at[0]], rows2, sg2).wait()
        pltpu.make_async_copy(rows1, acc.at[dst_v.at[0]], ss1).wait()
        pltpu.make_async_copy(rows3, acc.at[dst_v.at[0]], ss3).wait()
        plsc.subcore_barrier()
        pltpu.sync_copy(acc.at[pl.ds(s * RPT, RPT)],
                        out_hbm.at[pl.ds(c * NP + s * RPT, RPT)])

    return k


def _make_deg_kernel():
    @functools.partial(
        pl.kernel,
        out_type=jax.ShapeDtypeStruct((NC, NP, DDEG), jnp.float32),
        mesh=_get_mesh(),
        scratch_types=[
            pltpu.VMEM((NCHUNK, CH), jnp.int32),
            pltpu.VMEM((CH, DDEG), jnp.float32),
            pltpu.VMEM((CH, DDEG), jnp.float32),
            pltpu.VMEM_SHARED((NP, DDEG), jnp.float32),
        ],
    )
    def k(dst_hbm, out_hbm, dst_v, ones_v, zero_v, acc):
        c = lax.axis_index("c")
        s = lax.axis_index("s")
        wid = c * NS + s
        pltpu.sync_copy(dst_hbm.at[wid], dst_v)

        ov = jnp.full((16,), 1.0, jnp.float32)
        zv = jnp.zeros((16,), jnp.float32)

        def fbody(r, carry):
            ones_v[r, pl.ds(0, DDEG)] = ov
            zero_v[r, pl.ds(0, DDEG)] = zv
            return carry

        lax.fori_loop(0, CH, fbody, 0)
        for p in range(RPT // CH):
            pltpu.sync_copy(zero_v, acc.at[pl.ds(s * RPT + p * CH, CH)])
        plsc.subcore_barrier()

        def body(j, carry):
            pltpu.sync_copy(ones_v, acc.at[dst_v.at[j]], add=True)
            return carry

        lax.fori_loop(0, NCHUNK, body, 0)
        plsc.subcore_barrier()
        pltpu.sync_copy(acc.at[pl.ds(s * RPT, RPT)],
                        out_hbm.at[c, pl.ds(s * RPT, RPT)])

    return k


_BR = 512
_NB = NP // _BR


def _dinv_block(d0_ref, d1_ref):
    deg = d0_ref[:, 0:1] + d1_ref[:, 0:1] + 1.0
    return lax.rsqrt(deg)


def _tc_y1(x, W1s, d0, d1):
    H = DHID // 2

    def body(x_ref, w_ref, d0_ref, d1_ref, y_ref):
        dinv = _dinv_block(d0_ref, d1_ref)
        y_ref[...] = dinv * jnp.dot(x_ref[...], w_ref[0],
                                    preferred_element_type=jnp.float32)

    return pl.pallas_call(
        body,
        grid=(_NB, 2),
        in_specs=[
            pl.BlockSpec((_BR, DIN), lambda i, j: (i, 0)),
            pl.BlockSpec((1, DIN, H), lambda i, j: (j, 0, 0)),
            pl.BlockSpec((_BR, DDEG), lambda i, j: (i, 0)),
            pl.BlockSpec((_BR, DDEG), lambda i, j: (i, 0)),
        ],
        out_specs=pl.BlockSpec((_BR, H), lambda i, j: (j * _NB + i, 0)),
        out_shape=jax.ShapeDtypeStruct((NC * NP, H), jnp.float32),
    )(x, W1s, d0, d1)


def _tc_y2(s1f, y1f, d0, d1, b1, W2s):
    H1 = DHID // 2
    H2 = DOUT // 2

    def body(s1a_ref, s1b_ref, y1a_ref, y1b_ref, d0_ref, d1_ref, b1_ref,
             w_ref, y2_ref):
        dinv = _dinv_block(d0_ref, d1_ref)
        t = jnp.concatenate(
            [s1a_ref[...] + y1a_ref[...], s1b_ref[...] + y1b_ref[...]],
            axis=1)
        h = jax.nn.relu(dinv * t + b1_ref[...])
        y2_ref[...] = dinv * jnp.dot(h, w_ref[0],
                                     preferred_element_type=jnp.float32)

    return pl.pallas_call(
        body,
        grid=(_NB, 2),
        in_specs=[
            pl.BlockSpec((_BR, H1), lambda i, j: (i, 0)),
            pl.BlockSpec((_BR, H1), lambda i, j: (_NB + i, 0)),
            pl.BlockSpec((_BR, H1), lambda i, j: (i, 0)),
            pl.BlockSpec((_BR, H1), lambda i, j: (_NB + i, 0)),
            pl.BlockSpec((_BR, DDEG), lambda i, j: (i, 0)),
            pl.BlockSpec((_BR, DDEG), lambda i, j: (i, 0)),
            pl.BlockSpec((1, DHID), lambda i, j: (0, 0)),
            pl.BlockSpec((1, DHID, H2), lambda i, j: (j, 0, 0)),
        ],
        out_specs=pl.BlockSpec((_BR, H2), lambda i, j: (j * _NB + i, 0)),
        out_shape=jax.ShapeDtypeStruct((NC * NP, H2), jnp.float32),
    )(s1f, s1f, y1f, y1f, d0, d1, b1, W2s)


def _tc_final(s2f, y2f, d0, d1, b2, batch2d, Wlp, bl2d):
    H2 = DOUT // 2

    def body(s2a_ref, s2b_ref, y2a_ref, y2b_ref, d0_ref, d1_ref, b2_ref,
             bt_ref, wl_ref, bl_ref, out_ref, acc_ref):
        i = pl.program_id(0)

        @pl.when(i == 0)
        def _():
            acc_ref[...] = jnp.zeros_like(acc_ref)

        dinv = _dinv_block(d0_ref, d1_ref)
        t = jnp.concatenate(
            [s2a_ref[...] + y2a_ref[...], s2b_ref[...] + y2b_ref[...]],
            axis=1)
        h2 = dinv * t + b2_ref[...]
        iota = lax.broadcasted_iota(jnp.int32, (G, _BR), 0)
        oh = (bt_ref[...] == iota).astype(jnp.float32)
        acc_ref[:, 0:DOUT] = acc_ref[:, 0:DOUT] + jnp.dot(
            oh, h2, preferred_element_type=jnp.float32)
        acc_ref[:, DOUT:DOUT + 1] = (acc_ref[:, DOUT:DOUT + 1]
                                     + jnp.sum(oh, axis=1, keepdims=True))

        @pl.when(i == _NB - 1)
        def _():
            cnt = jnp.maximum(acc_ref[:, DOUT:DOUT + 1], 1.0)
            g = acc_ref[:, 0:DOUT] / cnt
            out_ref[...] = jnp.dot(g, wl_ref[...],
                                   preferred_element_type=jnp.float32) \
                + bl_ref[0, 0]

    return pl.pallas_call(
        body,
        grid=(_NB,),
        in_specs=[
            pl.BlockSpec((_BR, H2), lambda i: (i, 0)),
            pl.BlockSpec((_BR, H2), lambda i: (_NB + i, 0)),
            pl.BlockSpec((_BR, H2), lambda i: (i, 0)),
            pl.BlockSpec((_BR, H2), lambda i: (_NB + i, 0)),
            pl.BlockSpec((_BR, DDEG), lambda i: (i, 0)),
            pl.BlockSpec((_BR, DDEG), lambda i: (i, 0)),
            pl.BlockSpec((1, DOUT), lambda i: (0, 0)),
            pl.BlockSpec((1, _BR), lambda i: (0, i)),
            pl.BlockSpec((DOUT, 128), lambda i: (0, 0)),
            pl.BlockSpec((1, 1), lambda i: (0, 0)),
        ],
        out_specs=pl.BlockSpec((G, 128), lambda i: (0, 0)),
        out_shape=jax.ShapeDtypeStruct((G, 128), jnp.float32),
        scratch_shapes=[pltpu.VMEM((G, 128), jnp.float32)],
    )(s2f, s2f, y2f, y2f, d0, d1, b2, batch2d, Wlp, bl2d)


_sc_cache = {}


def _deg_kernel(dst_slab):
    if "deg" not in _sc_cache:
        _sc_cache["deg"] = _make_deg_kernel()
    return _sc_cache["deg"](dst_slab)


def _scatter_l1(src_slab, dst_slab, y):
    if 1 not in _sc_cache:
        _sc_cache[1] = _make_edge_scatter_cs(DHID // 2)
    return _sc_cache[1](src_slab, dst_slab, y)


def _scatter_l2(src_slab, dst_slab, y):
    if 2 not in _sc_cache:
        _sc_cache[2] = _make_edge_scatter_cs(DOUT // 2)
    return _sc_cache[2](src_slab, dst_slab, y)


def kernel(x, edge_index, batch, W1, b1, W2, b2, Wl, bl):
    src = edge_index[0]
    dst = edge_index[1]
    pad = jnp.full((EP - E,), N, dtype=jnp.int32)
    srcp = jnp.concatenate([src, pad])
    dstp = jnp.concatenate([dst, pad])
    dst_deg_slab = dstp.reshape(NW, NCHUNK, CH)
    src_base = srcp.reshape(NS, NCH2, CH)
    src_slab = jnp.stack([src_base, src_base + NP])
    dst_slab = dstp.reshape(NS, NCH2, CH)

    xp = jnp.zeros((NP, DIN), jnp.float32).at[:N].set(x)
    batchp = jnp.full((NP,), G, jnp.int32).at[:N].set(batch).reshape(1, NP)
    W1s = jnp.stack([W1[:, :DHID // 2], W1[:, DHID // 2:]])
    W2s = jnp.stack([W2[:, :DOUT // 2], W2[:, DOUT // 2:]])

    deg = _deg_kernel(dst_deg_slab)
    d0, d1 = deg[0], deg[1]

    y1f = _tc_y1(xp, W1s, d0, d1)
    s1f = _scatter_l1(src_slab, dst_slab, y1f)
    y2f = _tc_y2(s1f, y1f, d0, d1, b1.reshape(1, DHID), W2s)
    s2f = _scatter_l2(src_slab, dst_slab, y2f)

    Wlp = jnp.zeros((DOUT, 128), jnp.float32).at[:, 0].set(Wl[:, 0])
    out2 = _tc_final(s2f, y2f, d0, d1, b2.reshape(1, DOUT), batchp,
                     Wlp, bl.reshape(1, 1))
    return out2[:, 0:1]
